# baseline (speedup 1.0000x reference)
import functools

import jax
import jax.numpy as jnp
from jax.experimental import pallas as pl
from jax.experimental.pallas import tpu as pltpu


def _crf_kernel(feats_ref, trans_ref, seq_ref, out_ref,
                u_scr, r_scr, a_scr, *, tt_size, k_size):
    tb = pl.program_id(0)
    E = jnp.exp(trans_ref[...])
    sl1 = seq_ref[...] - 1
    t0 = tb * tt_size

    ones_row = jnp.ones((1, k_size), dtype=jnp.float32)

    def step(gt, carry):
        U, r, A, outv = carry
        expf = jnp.exp(feats_ref[gt - t0].astype(jnp.float32).T)
        grp = expf * r
        S = jnp.dot(E, U, preferred_element_type=jnp.float32)
        Un = S * grp
        s = jax.lax.dot_general(ones_row, Un, (((1,), (0,)), ((), ())),
                                preferred_element_type=jnp.float32)
        rn = 1.0 / s
        An = A + jnp.log(s)
        outv = jnp.where(sl1 == gt, An, outv)
        return Un, rn, An, outv

    @pl.when(tb == 0)
    def _():
        U0 = jnp.exp(feats_ref[0].astype(jnp.float32).T)
        s0 = jnp.sum(U0, axis=0, keepdims=True)
        r0 = 1.0 / s0
        A0 = jnp.log(s0)
        out0 = jnp.where(sl1 == 0, A0, jnp.zeros_like(A0))
        u_scr[...] = U0
        r_scr[...] = r0
        a_scr[...] = A0
        out_ref[...] = out0

    @pl.when(tb != 0)
    def _():
        carry = (u_scr[...], r_scr[...], a_scr[...], out_ref[...])
        U, r, A, outv = step(t0, carry)
        u_scr[...] = U
        r_scr[...] = r
        a_scr[...] = A
        out_ref[...] = outv

    carry = (u_scr[...], r_scr[...], a_scr[...], out_ref[...])
    for s in range(1, tt_size):
        carry = step(t0 + s, carry)
    U, r, A, outv = carry
    u_scr[...] = U
    r_scr[...] = r
    a_scr[...] = A
    out_ref[...] = outv


def kernel(feats, transitions, seq_lens):
    B, T, K = feats.shape
    TT = min(32, T)
    assert T % TT == 0
    ftbk = jnp.transpose(feats, (1, 0, 2))
    seq2 = seq_lens.reshape(1, B).astype(jnp.int32)
    out = pl.pallas_call(
        functools.partial(_crf_kernel, tt_size=TT, k_size=K),
        grid=(T // TT,),
        in_specs=[
            pl.BlockSpec((TT, B, K), lambda t: (t, 0, 0)),
            pl.BlockSpec((K, K), lambda t: (0, 0)),
            pl.BlockSpec((1, B), lambda t: (0, 0)),
        ],
        out_specs=pl.BlockSpec((1, B), lambda t: (0, 0)),
        out_shape=jax.ShapeDtypeStruct((1, B), jnp.float32),
        scratch_shapes=[
            pltpu.VMEM((K, B), jnp.float32),
            pltpu.VMEM((1, B), jnp.float32),
            pltpu.VMEM((1, B), jnp.float32),
        ],
        compiler_params=pltpu.CompilerParams(
            dimension_semantics=("arbitrary",),
            vmem_limit_bytes=50 * 1024 * 1024,
        ),
        name="crf_forward",
    )(ftbk, transitions, seq2)
    return out.reshape(B, 1)


# speedup vs baseline: 1.0541x; 1.0541x over previous
import functools

import jax
import jax.numpy as jnp
from jax.experimental import pallas as pl
from jax.experimental.pallas import tpu as pltpu


def _crf_kernel(feats_ref, trans_ref, seq_ref, out_ref,
                u_scr, r_scr, a_scr, *, tt_size, k_size):
    tb = pl.program_id(0)
    E = jnp.exp(trans_ref[...])
    sl1 = seq_ref[...] - 1
    t0 = tb * tt_size

    ones_row = jnp.ones((1, k_size), dtype=jnp.float32)

    def step(gt, carry):
        U, r, A, outv = carry
        expf = jnp.exp(feats_ref[gt - t0].astype(jnp.float32).T)
        grp = expf * r
        S = jnp.dot(E, U, preferred_element_type=jnp.float32)
        Un = S * grp
        s = jnp.sum(Un, axis=0, keepdims=True)
        rn = 1.0 / s
        An = A + jnp.log(s)
        outv = jnp.where(sl1 == gt, An, outv)
        return Un, rn, An, outv

    @pl.when(tb == 0)
    def _():
        U0 = jnp.exp(feats_ref[0].astype(jnp.float32).T)
        s0 = jnp.sum(U0, axis=0, keepdims=True)
        r0 = 1.0 / s0
        A0 = jnp.log(s0)
        out0 = jnp.where(sl1 == 0, A0, jnp.zeros_like(A0))
        u_scr[...] = U0
        r_scr[...] = r0
        a_scr[...] = A0
        out_ref[...] = out0

    @pl.when(tb != 0)
    def _():
        carry = (u_scr[...], r_scr[...], a_scr[...], out_ref[...])
        U, r, A, outv = step(t0, carry)
        u_scr[...] = U
        r_scr[...] = r
        a_scr[...] = A
        out_ref[...] = outv

    carry = (u_scr[...], r_scr[...], a_scr[...], out_ref[...])
    for s in range(1, tt_size):
        carry = step(t0 + s, carry)
    U, r, A, outv = carry
    u_scr[...] = U
    r_scr[...] = r
    a_scr[...] = A
    out_ref[...] = outv


def kernel(feats, transitions, seq_lens):
    B, T, K = feats.shape
    TT = min(32, T)
    assert T % TT == 0
    ftbk = jnp.transpose(feats, (1, 0, 2))
    seq2 = seq_lens.reshape(1, B).astype(jnp.int32)
    out = pl.pallas_call(
        functools.partial(_crf_kernel, tt_size=TT, k_size=K),
        grid=(T // TT,),
        in_specs=[
            pl.BlockSpec((TT, B, K), lambda t: (t, 0, 0)),
            pl.BlockSpec((K, K), lambda t: (0, 0)),
            pl.BlockSpec((1, B), lambda t: (0, 0)),
        ],
        out_specs=pl.BlockSpec((1, B), lambda t: (0, 0)),
        out_shape=jax.ShapeDtypeStruct((1, B), jnp.float32),
        scratch_shapes=[
            pltpu.VMEM((K, B), jnp.float32),
            pltpu.VMEM((1, B), jnp.float32),
            pltpu.VMEM((1, B), jnp.float32),
        ],
        compiler_params=pltpu.CompilerParams(
            dimension_semantics=("arbitrary",),
            vmem_limit_bytes=50 * 1024 * 1024,
        ),
        name="crf_forward",
    )(ftbk, transitions, seq2)
    return out.reshape(B, 1)
